# baseline (speedup 1.0000x reference)
import functools

import jax
import jax.numpy as jnp
from jax.experimental import pallas as pl
from jax.experimental.pallas import tpu as pltpu


def _se_block_kernel(x_ref, w1t_ref, w2t_ref, o_ref, *, inv_hw):
    x = x_ref[...]
    pooled = jnp.sum(x, axis=(2, 3), dtype=jnp.float32) * inv_hw
    hid = jnp.dot(pooled, w1t_ref[...], preferred_element_type=jnp.float32)
    hid = jnp.maximum(hid, 0.0)
    gate = jax.nn.sigmoid(
        jnp.dot(hid, w2t_ref[...], preferred_element_type=jnp.float32))
    o_ref[...] = x * gate[:, :, None, None].astype(x.dtype)


def _weight_spec(shape):
    try:
        return pl.BlockSpec(shape, lambda b: (0, 0), pipeline_mode=pl.Buffered(1))
    except (TypeError, AttributeError):
        return pl.BlockSpec(shape, lambda b: (0, 0))


def kernel(x, w1, w2):
    B, C, H, W = x.shape
    nmid = w1.shape[0]

    w1t = jnp.asarray(w1, jnp.float32).T
    w2t = jnp.asarray(w2, jnp.float32).T

    body = functools.partial(_se_block_kernel, inv_hw=1.0 / float(H * W))
    return pl.pallas_call(
        body,
        out_shape=jax.ShapeDtypeStruct((B, C, H, W), x.dtype),
        grid=(B,),
        in_specs=[
            pl.BlockSpec((1, C, H, W), lambda b: (b, 0, 0, 0)),
            _weight_spec((C, nmid)),
            _weight_spec((nmid, C)),
        ],
        out_specs=pl.BlockSpec((1, C, H, W), lambda b: (b, 0, 0, 0)),
        compiler_params=pltpu.CompilerParams(
            dimension_semantics=("parallel",),
            vmem_limit_bytes=64 << 20,
        ),
    )(x, w1t, w2t)


# speedup vs baseline: 6.4465x; 6.4465x over previous
import functools

import jax
import jax.numpy as jnp
from jax.experimental import pallas as pl
from jax.experimental.pallas import tpu as pltpu


def _se_block_kernel(x_ref, w1t_ref, w2t_ref, o_ref, *, inv_hw):
    x = x_ref[...]
    pooled = jnp.sum(x, axis=(1, 2), dtype=jnp.float32) * inv_hw
    hid = jnp.dot(pooled, w1t_ref[...], preferred_element_type=jnp.float32)
    hid = jnp.maximum(hid, 0.0)
    gate = jax.nn.sigmoid(
        jnp.dot(hid, w2t_ref[...], preferred_element_type=jnp.float32))
    o_ref[...] = x * gate[:, None, None, :].astype(x.dtype)


def _weight_spec(shape):
    try:
        return pl.BlockSpec(shape, lambda b: (0, 0), pipeline_mode=pl.Buffered(1))
    except (TypeError, AttributeError):
        return pl.BlockSpec(shape, lambda b: (0, 0))


def kernel(x, w1, w2):
    B, C, H, W = x.shape
    nmid = w1.shape[0]

    xt = jnp.transpose(x, (0, 2, 3, 1))
    w1t = jnp.asarray(w1, jnp.float32).T
    w2t = jnp.asarray(w2, jnp.float32).T

    body = functools.partial(_se_block_kernel, inv_hw=1.0 / float(H * W))
    out_t = pl.pallas_call(
        body,
        out_shape=jax.ShapeDtypeStruct((B, H, W, C), x.dtype),
        grid=(B,),
        in_specs=[
            pl.BlockSpec((1, H, W, C), lambda b: (b, 0, 0, 0)),
            _weight_spec((C, nmid)),
            _weight_spec((nmid, C)),
        ],
        out_specs=pl.BlockSpec((1, H, W, C), lambda b: (b, 0, 0, 0)),
        compiler_params=pltpu.CompilerParams(
            dimension_semantics=("parallel",),
            vmem_limit_bytes=64 << 20,
        ),
    )(xt, w1t, w2t)
    return jnp.transpose(out_t, (0, 3, 1, 2))


# speedup vs baseline: 6.9360x; 1.0759x over previous
import functools

import jax
import jax.numpy as jnp
from jax.experimental import pallas as pl
from jax.experimental.pallas import tpu as pltpu


def _se_block_kernel(x_ref, w1t_ref, w2t_ref, o_ref, *, inv_hw):
    x = x_ref[...]
    pooled = jnp.sum(x, axis=(1, 2), dtype=jnp.float32) * inv_hw
    hid = jnp.dot(pooled, w1t_ref[...], preferred_element_type=jnp.float32)
    hid = jnp.maximum(hid, 0.0)
    gate = jax.nn.sigmoid(
        jnp.dot(hid, w2t_ref[...], preferred_element_type=jnp.float32))
    o_ref[...] = x * gate[:, None, None, :].astype(x.dtype)


def _weight_spec(shape):
    try:
        return pl.BlockSpec(shape, lambda b: (0, 0), pipeline_mode=pl.Buffered(1))
    except (TypeError, AttributeError):
        return pl.BlockSpec(shape, lambda b: (0, 0))


def kernel(x, w1, w2):
    B, C, H, W = x.shape
    nmid = w1.shape[0]

    xt = jnp.transpose(x, (0, 2, 3, 1))
    w1t = jnp.asarray(w1, jnp.float32).T
    w2t = jnp.asarray(w2, jnp.float32).T

    tb = 2
    body = functools.partial(_se_block_kernel, inv_hw=1.0 / float(H * W))
    out_t = pl.pallas_call(
        body,
        out_shape=jax.ShapeDtypeStruct((B, H, W, C), x.dtype),
        grid=(B // tb,),
        in_specs=[
            pl.BlockSpec((tb, H, W, C), lambda b: (b, 0, 0, 0)),
            _weight_spec((C, nmid)),
            _weight_spec((nmid, C)),
        ],
        out_specs=pl.BlockSpec((tb, H, W, C), lambda b: (b, 0, 0, 0)),
        compiler_params=pltpu.CompilerParams(
            dimension_semantics=("parallel",),
            vmem_limit_bytes=64 << 20,
        ),
    )(xt, w1t, w2t)
    return jnp.transpose(out_t, (0, 3, 1, 2))


# speedup vs baseline: 7.1602x; 1.0323x over previous
import functools

import jax
import jax.numpy as jnp
from jax.experimental import pallas as pl
from jax.experimental.pallas import tpu as pltpu


def _se_block_kernel(x_ref, w1t_ref, w2t_ref, o_ref, *, inv_hw):
    x = x_ref[...]
    pooled = jnp.sum(x, axis=(1, 2), dtype=jnp.float32) * inv_hw
    hid = jnp.dot(pooled, w1t_ref[...], preferred_element_type=jnp.float32)
    hid = jnp.maximum(hid, 0.0)
    gate = jax.nn.sigmoid(
        jnp.dot(hid, w2t_ref[...], preferred_element_type=jnp.float32))
    o_ref[...] = x * gate[:, None, None, :].astype(x.dtype)


def _weight_spec(shape):
    try:
        return pl.BlockSpec(shape, lambda b: (0, 0), pipeline_mode=pl.Buffered(1))
    except (TypeError, AttributeError):
        return pl.BlockSpec(shape, lambda b: (0, 0))


def kernel(x, w1, w2):
    B, C, H, W = x.shape
    nmid = w1.shape[0]

    xt = jnp.transpose(x, (0, 2, 3, 1))
    w1t = jnp.asarray(w1, jnp.float32).T
    w2t = jnp.asarray(w2, jnp.float32).T

    tb = 4
    body = functools.partial(_se_block_kernel, inv_hw=1.0 / float(H * W))
    out_t = pl.pallas_call(
        body,
        out_shape=jax.ShapeDtypeStruct((B, H, W, C), x.dtype),
        grid=(B // tb,),
        in_specs=[
            pl.BlockSpec((tb, H, W, C), lambda b: (b, 0, 0, 0)),
            _weight_spec((C, nmid)),
            _weight_spec((nmid, C)),
        ],
        out_specs=pl.BlockSpec((tb, H, W, C), lambda b: (b, 0, 0, 0)),
        compiler_params=pltpu.CompilerParams(
            dimension_semantics=("parallel",),
            vmem_limit_bytes=64 << 20,
        ),
    )(xt, w1t, w2t)
    return jnp.transpose(out_t, (0, 3, 1, 2))
